# baseline (speedup 1.0000x reference)
import functools

import jax
import jax.numpy as jnp
from jax import lax
from jax.experimental import pallas as pl
from jax.experimental.pallas import tpu as pltpu

_TEMPERATURE = 0.8
_KMAX = 50


def _f32_to_key(x):
    i = lax.bitcast_convert_type(x, jnp.int32)
    return i ^ ((i >> 31) & jnp.int32(0x7FFFFFFF))


def _key_to_f32(k):
    i = k ^ ((k >> 31) & jnp.int32(0x7FFFFFFF))
    return lax.bitcast_convert_type(i, jnp.float32)


def _kth_largest_search(data, k_self, lo0, hi0):
    def cond(c):
        lo, hi = c
        return jnp.any(lo < hi)

    def step(c):
        lo, hi = c
        d = lax.bitcast_convert_type(hi - lo, jnp.uint32)
        half = lax.bitcast_convert_type((d + jnp.uint32(1)) >> 1, jnp.int32)
        mid = lo + half
        t = _key_to_f32(mid)
        cnt = jnp.sum(
            jnp.where(data >= t, jnp.float32(1.0), jnp.float32(0.0)),
            axis=1, keepdims=True)
        pred = cnt >= k_self
        return jnp.where(pred, mid, lo), jnp.where(pred, hi, mid - 1)

    lo, _ = lax.while_loop(cond, step, (lo0, hi0))
    return _key_to_f32(lo)


def _per_lane_top4(ypad):
    mx, mn = jnp.maximum, jnp.minimum
    a, b = ypad[:, :512], ypad[:, 512:]
    s1, s2 = mx(a, b), mn(a, b)
    a1, a2 = s1[:, :256], s2[:, :256]
    b1, b2 = s1[:, 256:], s2[:, 256:]
    o1, o4 = mx(a1, b1), mn(a2, b2)
    t1, t2 = mn(a1, b1), mx(a2, b2)
    lists = (o1, mx(t1, t2), mn(t1, t2), o4)
    n = 256
    while n > 1:
        h = n // 2
        a1, a2, a3, a4 = (l[:, :h] for l in lists)
        b1, b2, b3, b4 = (l[:, h:] for l in lists)
        h1, h2, h3, h4 = mx(a1, b4), mx(a2, b3), mx(a3, b2), mx(a4, b1)
        p1, p3 = mx(h1, h3), mn(h1, h3)
        p2, p4 = mx(h2, h4), mn(h2, h4)
        lists = (mx(p1, p2), mn(p1, p2), mx(p3, p4), mn(p3, p4))
        n = h
    return tuple(l[:, 0] for l in lists)


def _gumbel_bits(rows, cols, v):
    j = lax.bitcast_convert_type(rows * v + cols, jnp.uint32)
    ks0 = jnp.uint32(0)
    ks1 = jnp.uint32(1)
    ks2 = jnp.uint32(0x1BD11BDA) ^ ks0 ^ ks1
    x0 = jnp.zeros_like(j) + ks0
    x1 = j + ks1

    def rounds(x0, x1, rots):
        for r in rots:
            x0 = x0 + x1
            x1 = (x1 << jnp.uint32(r)) | (x1 >> jnp.uint32(32 - r))
            x1 = x1 ^ x0
        return x0, x1

    r0 = (13, 15, 26, 6)
    r1 = (17, 29, 16, 24)
    x0, x1 = rounds(x0, x1, r0)
    x0, x1 = x0 + ks1, x1 + ks2 + jnp.uint32(1)
    x0, x1 = rounds(x0, x1, r1)
    x0, x1 = x0 + ks2, x1 + ks0 + jnp.uint32(2)
    x0, x1 = rounds(x0, x1, r0)
    x0, x1 = x0 + ks0, x1 + ks1 + jnp.uint32(3)
    x0, x1 = rounds(x0, x1, r1)
    x0, x1 = x0 + ks1, x1 + ks2 + jnp.uint32(4)
    x0, x1 = rounds(x0, x1, r0)
    x0, x1 = x0 + ks2, x1 + ks0 + jnp.uint32(5)
    bits = x0 ^ x1

    fb = (bits >> jnp.uint32(9)) | jnp.uint32(0x3F800000)
    f = lax.bitcast_convert_type(fb, jnp.float32) - jnp.float32(1.0)
    tiny = jnp.float32(jnp.finfo(jnp.float32).tiny)
    u = jnp.maximum(tiny, f * (jnp.float32(1.0) - tiny) + tiny)
    return -jnp.log(-jnp.log(u))


def _make_body(br, v):
    def _body(tk_ref, x_ref, probs_ref, idx_ref):
        y = x_ref[...] / jnp.float32(_TEMPERATURE)
        k_sel = jnp.clip(tk_ref[0], 1, _KMAX)
        k_self = k_sel.astype(jnp.float32)
        neg_inf = jnp.float32(-jnp.inf)

        vfull = (v // 128) * 128
        ya = y[:, :vfull].reshape(br, vfull // 128, 128)
        rem = v - vfull
        if rem:
            yb = jnp.concatenate(
                [y[:, vfull:],
                 jnp.full((br, 128 - rem), neg_inf, jnp.float32)],
                axis=1,
            ).reshape(br, 1, 128)
            ya = jnp.concatenate([ya, yb], axis=1)
        npad = 1024 - ya.shape[1]
        ypad = jnp.concatenate(
            [ya, jnp.full((br, npad, 128), neg_inf, jnp.float32)], axis=1
        )
        m1, m2, m3, m4 = _per_lane_top4(ypad)
        cand = jnp.concatenate([m1, m2, m3, m4], axis=1)
        rmax = jnp.max(m1, axis=1, keepdims=True)

        lo0 = _f32_to_key(jnp.min(m4, axis=1, keepdims=True))
        hi0 = _f32_to_key(rmax)
        t_c = _kth_largest_search(cand, k_self, lo0, hi0)

        cnt_gt = jnp.sum(
            jnp.where(y > t_c, jnp.float32(1.0), jnp.float32(0.0)),
            axis=1, keepdims=True)
        ok = cnt_gt < k_self

        thresh = lax.cond(
            jnp.all(ok),
            lambda: t_c,
            lambda: _kth_largest_search(y, k_self, lo0, hi0),
        )

        keep = y >= thresh
        s = jnp.where(keep, jnp.exp(y - rmax), jnp.float32(0.0))
        denom = jnp.sum(s, axis=1, keepdims=True)
        probs_ref[...] = s * (jnp.float32(1.0) / denom)

        cols = lax.broadcasted_iota(jnp.int32, (br, v), 1)
        rows = pl.program_id(0) * br + lax.broadcasted_iota(
            jnp.int32, (br, v), 0)
        g = _gumbel_bits(rows, cols, v)
        z = jnp.where(keep, y + g, neg_inf)
        zmax = jnp.max(z, axis=1, keepdims=True)
        idx = jnp.min(
            jnp.where(z == zmax, cols, jnp.int32(0x7FFFFFFF)), axis=1)
        idx_ref[...] = idx.reshape(br, 1)

    return _body


@jax.jit
def kernel(logits, top_k):
    r, v = logits.shape
    br = 8
    tk = jnp.asarray(top_k, jnp.int32).reshape(1)

    grid_spec = pltpu.PrefetchScalarGridSpec(
        num_scalar_prefetch=1,
        grid=(r // br,),
        in_specs=[
            pl.BlockSpec((br, v), lambda i, tk_ref: (i, 0)),
        ],
        out_specs=[
            pl.BlockSpec((br, v), lambda i, tk_ref: (i, 0)),
            pl.BlockSpec((br, 1), lambda i, tk_ref: (i, 0)),
        ],
    )
    probs, idx = pl.pallas_call(
        _make_body(br, v),
        grid_spec=grid_spec,
        out_shape=[
            jax.ShapeDtypeStruct((r, v), jnp.float32),
            jax.ShapeDtypeStruct((r, 1), jnp.int32),
        ],
    )(tk, logits)
    return probs, idx[:, 0]


# speedup vs baseline: 1.8705x; 1.8705x over previous
import functools

import jax
import jax.numpy as jnp
from jax import lax
from jax.experimental import pallas as pl
from jax.experimental.pallas import tpu as pltpu

_TEMPERATURE = 0.8
_KMAX = 50


def _f32_to_key(x):
    i = lax.bitcast_convert_type(x, jnp.int32)
    return i ^ ((i >> 31) & jnp.int32(0x7FFFFFFF))


def _key_to_f32(k):
    i = k ^ ((k >> 31) & jnp.int32(0x7FFFFFFF))
    return lax.bitcast_convert_type(i, jnp.float32)


def _kth_largest_search(data, k_self, lo0, hi0):
    def cond(c):
        lo, hi = c
        return jnp.any(lo < hi)

    def step(c):
        lo, hi = c
        d = lax.bitcast_convert_type(hi - lo, jnp.uint32)
        half = lax.bitcast_convert_type((d + jnp.uint32(1)) >> 1, jnp.int32)
        mid = lo + half
        t = _key_to_f32(mid)
        cnt = jnp.sum(
            jnp.where(data >= t, jnp.float32(1.0), jnp.float32(0.0)),
            axis=1, keepdims=True)
        pred = cnt >= k_self
        return jnp.where(pred, mid, lo), jnp.where(pred, hi, mid - 1)

    lo, _ = lax.while_loop(cond, step, (lo0, hi0))
    return _key_to_f32(lo)


def _sel(c, a, b):
    return jnp.where(c, a, b)


def _per_lane_top4_idx(ypad, qpad):
    av, bv = ypad[:, :512], ypad[:, 512:]
    aq, bq = qpad[:, :512], qpad[:, 512:]
    c = av >= bv
    s1v, s1q = _sel(c, av, bv), _sel(c, aq, bq)
    s2v, s2q = _sel(c, bv, av), _sel(c, bq, aq)
    a1v, a1q = s1v[:, :256], s1q[:, :256]
    a2v, a2q = s2v[:, :256], s2q[:, :256]
    b1v, b1q = s1v[:, 256:], s1q[:, 256:]
    b2v, b2q = s2v[:, 256:], s2q[:, 256:]
    c1 = a1v >= b1v
    o1v, o1q = _sel(c1, a1v, b1v), _sel(c1, a1q, b1q)
    t1v, t1q = _sel(c1, b1v, a1v), _sel(c1, b1q, a1q)
    c2 = a2v >= b2v
    t2v, t2q = _sel(c2, a2v, b2v), _sel(c2, a2q, b2q)
    o4v, o4q = _sel(c2, b2v, a2v), _sel(c2, b2q, a2q)
    c3 = t1v >= t2v
    o2v, o2q = _sel(c3, t1v, t2v), _sel(c3, t1q, t2q)
    o3v, o3q = _sel(c3, t2v, t1v), _sel(c3, t2q, t1q)
    lists = ((o1v, o1q), (o2v, o2q), (o3v, o3q), (o4v, o4q))
    n = 256
    while n > 1:
        h = n // 2
        a = [(v[:, :h], q[:, :h]) for v, q in lists]
        b = [(v[:, h:], q[:, h:]) for v, q in lists]

        def hi(x, y):
            c = x[0] >= y[0]
            return _sel(c, x[0], y[0]), _sel(c, x[1], y[1])

        def hilo(x, y):
            c = x[0] >= y[0]
            return ((_sel(c, x[0], y[0]), _sel(c, x[1], y[1])),
                    (_sel(c, y[0], x[0]), _sel(c, y[1], x[1])))

        h1 = hi(a[0], b[3])
        h2 = hi(a[1], b[2])
        h3 = hi(a[2], b[1])
        h4 = hi(a[3], b[0])
        p1, p3 = hilo(h1, h3)
        p2, p4 = hilo(h2, h4)
        q1, q2 = hilo(p1, p2)
        q3, q4 = hilo(p3, p4)
        lists = (q1, q2, q3, q4)
        n = h
    return tuple((v[:, 0], q[:, 0]) for v, q in lists)


def _gumbel_at(rows, cols, v):
    j = lax.bitcast_convert_type(rows * v + cols, jnp.uint32)
    ks0 = jnp.uint32(0)
    ks1 = jnp.uint32(1)
    ks2 = jnp.uint32(0x1BD11BDA) ^ ks0 ^ ks1
    x0 = jnp.zeros_like(j) + ks0
    x1 = j + ks1

    def rounds(x0, x1, rots):
        for r in rots:
            x0 = x0 + x1
            x1 = (x1 << jnp.uint32(r)) | (x1 >> jnp.uint32(32 - r))
            x1 = x1 ^ x0
        return x0, x1

    r0 = (13, 15, 26, 6)
    r1 = (17, 29, 16, 24)
    x0, x1 = rounds(x0, x1, r0)
    x0, x1 = x0 + ks1, x1 + ks2 + jnp.uint32(1)
    x0, x1 = rounds(x0, x1, r1)
    x0, x1 = x0 + ks2, x1 + ks0 + jnp.uint32(2)
    x0, x1 = rounds(x0, x1, r0)
    x0, x1 = x0 + ks0, x1 + ks1 + jnp.uint32(3)
    x0, x1 = rounds(x0, x1, r1)
    x0, x1 = x0 + ks1, x1 + ks2 + jnp.uint32(4)
    x0, x1 = rounds(x0, x1, r0)
    x0, x1 = x0 + ks2, x1 + ks0 + jnp.uint32(5)
    bits = x0 ^ x1

    fb = (bits >> jnp.uint32(9)) | jnp.uint32(0x3F800000)
    f = lax.bitcast_convert_type(fb, jnp.float32) - jnp.float32(1.0)
    tiny = jnp.float32(jnp.finfo(jnp.float32).tiny)
    u = jnp.maximum(tiny, f * (jnp.float32(1.0) - tiny) + tiny)
    return -jnp.log(-jnp.log(u))


def _argmin_idx(z, zmax, colidx):
    big = jnp.int32(0x7FFFFFFF)
    return jnp.min(jnp.where(z == zmax, colidx, big), axis=1)


def _make_body(br, v):
    def _body(tk_ref, x_ref, probs_ref, idx_ref):
        y = x_ref[...] / jnp.float32(_TEMPERATURE)
        k_sel = jnp.clip(tk_ref[0], 1, _KMAX)
        k_self = k_sel.astype(jnp.float32)
        neg_inf = jnp.float32(-jnp.inf)
        row_base = pl.program_id(0) * br

        vfull = (v // 128) * 128
        ya = y[:, :vfull].reshape(br, vfull // 128, 128)
        rem = v - vfull
        if rem:
            yb = jnp.concatenate(
                [y[:, vfull:],
                 jnp.full((br, 128 - rem), neg_inf, jnp.float32)],
                axis=1,
            ).reshape(br, 1, 128)
            ya = jnp.concatenate([ya, yb], axis=1)
        npad = 1024 - ya.shape[1]
        ypad = jnp.concatenate(
            [ya, jnp.full((br, npad, 128), neg_inf, jnp.float32)], axis=1
        )
        qpad = lax.broadcasted_iota(jnp.int32, (br, 1024, 128), 1)
        tops = _per_lane_top4_idx(ypad, qpad)
        (m1, q1), (m2, q2), (m3, q3), (m4, q4) = tops
        cand = jnp.concatenate([m1, m2, m3, m4], axis=1)
        qcand = jnp.concatenate([q1, q2, q3, q4], axis=1)
        lane = lax.broadcasted_iota(jnp.int32, (br, 512), 1) & jnp.int32(127)
        col_cand = qcand * jnp.int32(128) + lane
        rmax = jnp.max(m1, axis=1, keepdims=True)

        lo0 = _f32_to_key(jnp.min(m4, axis=1, keepdims=True))
        hi0 = _f32_to_key(rmax)
        t_c = _kth_largest_search(cand, k_self, lo0, hi0)

        one, zero = jnp.float32(1.0), jnp.float32(0.0)
        keep0 = y >= t_c
        cnt_gt = jnp.sum(jnp.where(y > t_c, one, zero), axis=1, keepdims=True)
        cnt_ge = jnp.sum(jnp.where(keep0, one, zero), axis=1, keepdims=True)
        s0 = jnp.where(keep0, jnp.exp(y - rmax), zero)
        denom0 = jnp.sum(s0, axis=1, keepdims=True)
        ccov = jnp.sum(jnp.where(cand >= t_c, one, zero), axis=1,
                       keepdims=True)
        ok = jnp.all((cnt_gt < k_self) & (cnt_ge == ccov))

        def fast():
            probs_ref[...] = s0 * (one / denom0)
            rows = row_base + lax.broadcasted_iota(jnp.int32, (br, 512), 0)
            g = _gumbel_at(rows, col_cand, v)
            z = jnp.where(cand >= t_c, cand + g, neg_inf)
            zmax = jnp.max(z, axis=1, keepdims=True)
            idx_ref[...] = _argmin_idx(z, zmax, col_cand).reshape(br, 1)

        def slow():
            thresh = _kth_largest_search(y, k_self, lo0, hi0)
            keep = y >= thresh
            s = jnp.where(keep, jnp.exp(y - rmax), zero)
            denom = jnp.sum(s, axis=1, keepdims=True)
            probs_ref[...] = s * (one / denom)
            cols = lax.broadcasted_iota(jnp.int32, (br, v), 1)
            rows = row_base + lax.broadcasted_iota(jnp.int32, (br, v), 0)
            g = _gumbel_at(rows, cols, v)
            z = jnp.where(keep, y + g, neg_inf)
            zmax = jnp.max(z, axis=1, keepdims=True)
            idx_ref[...] = _argmin_idx(z, zmax, cols).reshape(br, 1)

        lax.cond(ok, fast, slow)

    return _body


@jax.jit
def kernel(logits, top_k):
    r, v = logits.shape
    br = 8
    tk = jnp.asarray(top_k, jnp.int32).reshape(1)

    grid_spec = pltpu.PrefetchScalarGridSpec(
        num_scalar_prefetch=1,
        grid=(r // br,),
        in_specs=[
            pl.BlockSpec((br, v), lambda i, tk_ref: (i, 0)),
        ],
        out_specs=[
            pl.BlockSpec((br, v), lambda i, tk_ref: (i, 0)),
            pl.BlockSpec((br, 1), lambda i, tk_ref: (i, 0)),
        ],
    )
    probs, idx = pl.pallas_call(
        _make_body(br, v),
        grid_spec=grid_spec,
        out_shape=[
            jax.ShapeDtypeStruct((r, v), jnp.float32),
            jax.ShapeDtypeStruct((r, 1), jnp.int32),
        ],
    )(tk, logits)
    return probs, idx[:, 0]


# speedup vs baseline: 1.9015x; 1.0166x over previous
import functools

import jax
import jax.numpy as jnp
from jax import lax
from jax.experimental import pallas as pl
from jax.experimental.pallas import tpu as pltpu

_TEMPERATURE = 0.8
_KMAX = 50


def _f32_to_key(x):
    i = lax.bitcast_convert_type(x, jnp.int32)
    return i ^ ((i >> 31) & jnp.int32(0x7FFFFFFF))


def _key_to_f32(k):
    i = k ^ ((k >> 31) & jnp.int32(0x7FFFFFFF))
    return lax.bitcast_convert_type(i, jnp.float32)


def _kth_largest_search(data, k_self, lo0, hi0):
    def cond(c):
        lo, hi = c
        return jnp.any(lo < hi)

    def step(c):
        lo, hi = c
        d = lax.bitcast_convert_type(hi - lo, jnp.uint32)
        half = lax.bitcast_convert_type((d + jnp.uint32(1)) >> 1, jnp.int32)
        mid = lo + half
        t = _key_to_f32(mid)
        cnt = jnp.sum(
            jnp.where(data >= t, jnp.float32(1.0), jnp.float32(0.0)),
            axis=1, keepdims=True)
        pred = cnt >= k_self
        return jnp.where(pred, mid, lo), jnp.where(pred, hi, mid - 1)

    lo, _ = lax.while_loop(cond, step, (lo0, hi0))
    return _key_to_f32(lo)


def _sel(c, a, b):
    return jnp.where(c, a, b)


def _per_lane_top4_idx(ypad, qpad):
    n0 = ypad.shape[1]
    h1, h2 = n0 // 2, n0 // 4
    av, bv = ypad[:, :h1], ypad[:, h1:]
    aq, bq = qpad[:, :h1], qpad[:, h1:]
    c = av >= bv
    s1v, s1q = _sel(c, av, bv), _sel(c, aq, bq)
    s2v, s2q = _sel(c, bv, av), _sel(c, bq, aq)
    a1v, a1q = s1v[:, :h2], s1q[:, :h2]
    a2v, a2q = s2v[:, :h2], s2q[:, :h2]
    b1v, b1q = s1v[:, h2:], s1q[:, h2:]
    b2v, b2q = s2v[:, h2:], s2q[:, h2:]
    c1 = a1v >= b1v
    o1v, o1q = _sel(c1, a1v, b1v), _sel(c1, a1q, b1q)
    t1v, t1q = _sel(c1, b1v, a1v), _sel(c1, b1q, a1q)
    c2 = a2v >= b2v
    t2v, t2q = _sel(c2, a2v, b2v), _sel(c2, a2q, b2q)
    o4v, o4q = _sel(c2, b2v, a2v), _sel(c2, b2q, a2q)
    c3 = t1v >= t2v
    o2v, o2q = _sel(c3, t1v, t2v), _sel(c3, t1q, t2q)
    o3v, o3q = _sel(c3, t2v, t1v), _sel(c3, t2q, t1q)
    lists = ((o1v, o1q), (o2v, o2q), (o3v, o3q), (o4v, o4q))
    n = h2
    while n > 1:
        h = n // 2
        a = [(v[:, :h], q[:, :h]) for v, q in lists]
        b = [(v[:, h:2 * h], q[:, h:2 * h]) for v, q in lists]
        carry = None
        if n % 2:
            carry = [(v[:, 2 * h:], q[:, 2 * h:]) for v, q in lists]

        def hi(x, y):
            c = x[0] >= y[0]
            return _sel(c, x[0], y[0]), _sel(c, x[1], y[1])

        def hilo(x, y):
            c = x[0] >= y[0]
            return ((_sel(c, x[0], y[0]), _sel(c, x[1], y[1])),
                    (_sel(c, y[0], x[0]), _sel(c, y[1], x[1])))

        m1 = hi(a[0], b[3])
        m2 = hi(a[1], b[2])
        m3 = hi(a[2], b[1])
        m4 = hi(a[3], b[0])
        p1, p3 = hilo(m1, m3)
        p2, p4 = hilo(m2, m4)
        q1, q2 = hilo(p1, p2)
        q3, q4 = hilo(p3, p4)
        lists = (q1, q2, q3, q4)
        if carry is not None:
            lists = tuple(
                (jnp.concatenate([lv, cv], axis=1),
                 jnp.concatenate([lq, cq], axis=1))
                for (lv, lq), (cv, cq) in zip(lists, carry))
            n = h + 1
        else:
            n = h
    return tuple((v[:, 0], q[:, 0]) for v, q in lists)


def _gumbel_at(rows, cols, v):
    j = lax.bitcast_convert_type(rows * v + cols, jnp.uint32)
    ks0 = jnp.uint32(0)
    ks1 = jnp.uint32(1)
    ks2 = jnp.uint32(0x1BD11BDA) ^ ks0 ^ ks1
    x0 = jnp.zeros_like(j) + ks0
    x1 = j + ks1

    def rounds(x0, x1, rots):
        for r in rots:
            x0 = x0 + x1
            x1 = (x1 << jnp.uint32(r)) | (x1 >> jnp.uint32(32 - r))
            x1 = x1 ^ x0
        return x0, x1

    r0 = (13, 15, 26, 6)
    r1 = (17, 29, 16, 24)
    x0, x1 = rounds(x0, x1, r0)
    x0, x1 = x0 + ks1, x1 + ks2 + jnp.uint32(1)
    x0, x1 = rounds(x0, x1, r1)
    x0, x1 = x0 + ks2, x1 + ks0 + jnp.uint32(2)
    x0, x1 = rounds(x0, x1, r0)
    x0, x1 = x0 + ks0, x1 + ks1 + jnp.uint32(3)
    x0, x1 = rounds(x0, x1, r1)
    x0, x1 = x0 + ks1, x1 + ks2 + jnp.uint32(4)
    x0, x1 = rounds(x0, x1, r0)
    x0, x1 = x0 + ks2, x1 + ks0 + jnp.uint32(5)
    bits = x0 ^ x1

    fb = (bits >> jnp.uint32(9)) | jnp.uint32(0x3F800000)
    f = lax.bitcast_convert_type(fb, jnp.float32) - jnp.float32(1.0)
    tiny = jnp.float32(jnp.finfo(jnp.float32).tiny)
    u = jnp.maximum(tiny, f * (jnp.float32(1.0) - tiny) + tiny)
    return -jnp.log(-jnp.log(u))


def _argmin_idx(z, zmax, colidx):
    big = jnp.int32(0x7FFFFFFF)
    return jnp.min(jnp.where(z == zmax, colidx, big), axis=1)


def _make_body(br, v):
    def _body(tk_ref, x_ref, probs_ref, idx_ref):
        y = x_ref[...] / jnp.float32(_TEMPERATURE)
        k_sel = jnp.clip(tk_ref[0], 1, _KMAX)
        k_self = k_sel.astype(jnp.float32)
        neg_inf = jnp.float32(-jnp.inf)
        row_base = pl.program_id(0) * br

        vfull = (v // 128) * 128
        ya = y[:, :vfull].reshape(br, vfull // 128, 128)
        rem = v - vfull
        if rem:
            yb = jnp.concatenate(
                [y[:, vfull:],
                 jnp.full((br, 128 - rem), neg_inf, jnp.float32)],
                axis=1,
            ).reshape(br, 1, 128)
            ya = jnp.concatenate([ya, yb], axis=1)
        ntree = -(-ya.shape[1] // 4) * 4
        npad = ntree - ya.shape[1]
        if npad:
            ypad = jnp.concatenate(
                [ya, jnp.full((br, npad, 128), neg_inf, jnp.float32)],
                axis=1)
        else:
            ypad = ya
        qpad = lax.broadcasted_iota(jnp.int32, (br, ntree, 128), 1)
        tops = _per_lane_top4_idx(ypad, qpad)
        (m1, q1), (m2, q2), (m3, q3), (m4, q4) = tops
        cand = jnp.concatenate([m1, m2, m3, m4], axis=1)
        qcand = jnp.concatenate([q1, q2, q3, q4], axis=1)
        lane = lax.broadcasted_iota(jnp.int32, (br, 512), 1) & jnp.int32(127)
        col_cand = qcand * jnp.int32(128) + lane
        rmax = jnp.max(m1, axis=1, keepdims=True)

        lo0 = _f32_to_key(jnp.min(m4, axis=1, keepdims=True))
        hi0 = _f32_to_key(rmax)
        t_c = _kth_largest_search(cand, k_self, lo0, hi0)

        one, zero = jnp.float32(1.0), jnp.float32(0.0)
        keep0 = y >= t_c
        cnt_gt = jnp.sum(jnp.where(y > t_c, one, zero), axis=1, keepdims=True)
        cnt_ge = jnp.sum(jnp.where(keep0, one, zero), axis=1, keepdims=True)
        s0 = jnp.where(keep0, jnp.exp(y - rmax), zero)
        denom0 = jnp.sum(s0, axis=1, keepdims=True)
        ccov = jnp.sum(jnp.where(cand >= t_c, one, zero), axis=1,
                       keepdims=True)
        ok = jnp.all((cnt_gt < k_self) & (cnt_ge == ccov))

        def fast():
            probs_ref[...] = s0 * (one / denom0)
            rows = row_base + lax.broadcasted_iota(jnp.int32, (br, 512), 0)
            g = _gumbel_at(rows, col_cand, v)
            z = jnp.where(cand >= t_c, cand + g, neg_inf)
            zmax = jnp.max(z, axis=1, keepdims=True)
            idx_ref[...] = _argmin_idx(z, zmax, col_cand).reshape(br, 1)

        def slow():
            thresh = _kth_largest_search(y, k_self, lo0, hi0)
            keep = y >= thresh
            s = jnp.where(keep, jnp.exp(y - rmax), zero)
            denom = jnp.sum(s, axis=1, keepdims=True)
            probs_ref[...] = s * (one / denom)
            cols = lax.broadcasted_iota(jnp.int32, (br, v), 1)
            rows = row_base + lax.broadcasted_iota(jnp.int32, (br, v), 0)
            g = _gumbel_at(rows, cols, v)
            z = jnp.where(keep, y + g, neg_inf)
            zmax = jnp.max(z, axis=1, keepdims=True)
            idx_ref[...] = _argmin_idx(z, zmax, cols).reshape(br, 1)

        lax.cond(ok, fast, slow)

    return _body


@jax.jit
def kernel(logits, top_k):
    r, v = logits.shape
    br = 8
    tk = jnp.asarray(top_k, jnp.int32).reshape(1)

    grid_spec = pltpu.PrefetchScalarGridSpec(
        num_scalar_prefetch=1,
        grid=(r // br,),
        in_specs=[
            pl.BlockSpec((br, v), lambda i, tk_ref: (i, 0)),
        ],
        out_specs=[
            pl.BlockSpec((br, v), lambda i, tk_ref: (i, 0)),
            pl.BlockSpec((br, 1), lambda i, tk_ref: (i, 0)),
        ],
    )
    probs, idx = pl.pallas_call(
        _make_body(br, v),
        grid_spec=grid_spec,
        out_shape=[
            jax.ShapeDtypeStruct((r, v), jnp.float32),
            jax.ShapeDtypeStruct((r, 1), jnp.int32),
        ],
    )(tk, logits)
    return probs, idx[:, 0]


# speedup vs baseline: 1.9437x; 1.0222x over previous
import functools

import jax
import jax.numpy as jnp
from jax import lax
from jax.experimental import pallas as pl
from jax.experimental.pallas import tpu as pltpu

_TEMPERATURE = 0.8
_KMAX = 50


def _f32_to_key(x):
    i = lax.bitcast_convert_type(x, jnp.int32)
    return i ^ ((i >> 31) & jnp.int32(0x7FFFFFFF))


def _key_to_f32(k):
    i = k ^ ((k >> 31) & jnp.int32(0x7FFFFFFF))
    return lax.bitcast_convert_type(i, jnp.float32)


def _kth_largest_search(data, k_self, lo0, hi0):
    def cond(c):
        lo, hi = c
        return jnp.any(lo < hi)

    def step(c):
        lo, hi = c
        d = lax.bitcast_convert_type(hi - lo, jnp.uint32)
        half = lax.bitcast_convert_type((d + jnp.uint32(1)) >> 1, jnp.int32)
        mid = lo + half
        t = _key_to_f32(mid)
        cnt = jnp.sum(
            jnp.where(data >= t, jnp.float32(1.0), jnp.float32(0.0)),
            axis=1, keepdims=True)
        pred = cnt >= k_self
        return jnp.where(pred, mid, lo), jnp.where(pred, hi, mid - 1)

    lo, _ = lax.while_loop(cond, step, (lo0, hi0))
    return _key_to_f32(lo)


def _sel(c, a, b):
    return jnp.where(c, a, b)


def _per_lane_top4_idx(ypad, qpad):
    n0 = ypad.shape[1]
    h1, h2 = n0 // 2, n0 // 4
    av, bv = ypad[:, :h1], ypad[:, h1:]
    aq, bq = qpad[:, :h1], qpad[:, h1:]
    c = av >= bv
    s1v, s1q = _sel(c, av, bv), _sel(c, aq, bq)
    s2v, s2q = _sel(c, bv, av), _sel(c, bq, aq)
    a1v, a1q = s1v[:, :h2], s1q[:, :h2]
    a2v, a2q = s2v[:, :h2], s2q[:, :h2]
    b1v, b1q = s1v[:, h2:], s1q[:, h2:]
    b2v, b2q = s2v[:, h2:], s2q[:, h2:]
    c1 = a1v >= b1v
    o1v, o1q = _sel(c1, a1v, b1v), _sel(c1, a1q, b1q)
    t1v, t1q = _sel(c1, b1v, a1v), _sel(c1, b1q, a1q)
    c2 = a2v >= b2v
    t2v, t2q = _sel(c2, a2v, b2v), _sel(c2, a2q, b2q)
    o4v, o4q = _sel(c2, b2v, a2v), _sel(c2, b2q, a2q)
    c3 = t1v >= t2v
    o2v, o2q = _sel(c3, t1v, t2v), _sel(c3, t1q, t2q)
    o3v, o3q = _sel(c3, t2v, t1v), _sel(c3, t2q, t1q)
    lists = ((o1v, o1q), (o2v, o2q), (o3v, o3q), (o4v, o4q))
    n = h2
    while n > 1:
        h = n // 2
        a = [(v[:, :h], q[:, :h]) for v, q in lists]
        b = [(v[:, h:2 * h], q[:, h:2 * h]) for v, q in lists]
        carry = None
        if n % 2:
            carry = [(v[:, 2 * h:], q[:, 2 * h:]) for v, q in lists]
        lists = _merge4(a, b)
        if carry is not None:
            lists = tuple(
                (jnp.concatenate([lv, cv], axis=1),
                 jnp.concatenate([lq, cq], axis=1))
                for (lv, lq), (cv, cq) in zip(lists, carry))
            n = h + 1
        else:
            n = h
    return tuple((v[:, 0], q[:, 0]) for v, q in lists)


def _merge4(a, b):
    def hi(x, y):
        c = x[0] >= y[0]
        return _sel(c, x[0], y[0]), _sel(c, x[1], y[1])

    def hilo(x, y):
        c = x[0] >= y[0]
        return ((_sel(c, x[0], y[0]), _sel(c, x[1], y[1])),
                (_sel(c, y[0], x[0]), _sel(c, y[1], x[1])))

    m1 = hi(a[0], b[3])
    m2 = hi(a[1], b[2])
    m3 = hi(a[2], b[1])
    m4 = hi(a[3], b[0])
    p1, p3 = hilo(m1, m3)
    p2, p4 = hilo(m2, m4)
    q1, q2 = hilo(p1, p2)
    q3, q4 = hilo(p3, p4)
    return (q1, q2, q3, q4)


def _gumbel_at(rows, cols, v):
    j = lax.bitcast_convert_type(rows * v + cols, jnp.uint32)
    ks0 = jnp.uint32(0)
    ks1 = jnp.uint32(1)
    ks2 = jnp.uint32(0x1BD11BDA) ^ ks0 ^ ks1
    x0 = jnp.zeros_like(j) + ks0
    x1 = j + ks1

    def rounds(x0, x1, rots):
        for r in rots:
            x0 = x0 + x1
            x1 = (x1 << jnp.uint32(r)) | (x1 >> jnp.uint32(32 - r))
            x1 = x1 ^ x0
        return x0, x1

    r0 = (13, 15, 26, 6)
    r1 = (17, 29, 16, 24)
    x0, x1 = rounds(x0, x1, r0)
    x0, x1 = x0 + ks1, x1 + ks2 + jnp.uint32(1)
    x0, x1 = rounds(x0, x1, r1)
    x0, x1 = x0 + ks2, x1 + ks0 + jnp.uint32(2)
    x0, x1 = rounds(x0, x1, r0)
    x0, x1 = x0 + ks0, x1 + ks1 + jnp.uint32(3)
    x0, x1 = rounds(x0, x1, r1)
    x0, x1 = x0 + ks1, x1 + ks2 + jnp.uint32(4)
    x0, x1 = rounds(x0, x1, r0)
    x0, x1 = x0 + ks2, x1 + ks0 + jnp.uint32(5)
    bits = x0 ^ x1

    fb = (bits >> jnp.uint32(9)) | jnp.uint32(0x3F800000)
    f = lax.bitcast_convert_type(fb, jnp.float32) - jnp.float32(1.0)
    tiny = jnp.float32(jnp.finfo(jnp.float32).tiny)
    u = jnp.maximum(tiny, f * (jnp.float32(1.0) - tiny) + tiny)
    return -jnp.log(-jnp.log(u))


def _argmin_idx(z, zmax, colidx):
    big = jnp.int32(0x7FFFFFFF)
    return jnp.min(jnp.where(z == zmax, colidx, big), axis=1)


def _make_body(br, v):
    def _body(tk_ref, x_ref, probs_ref, idx_ref):
        y = x_ref[...] / jnp.float32(_TEMPERATURE)
        k_sel = jnp.clip(tk_ref[0], 1, _KMAX)
        k_self = k_sel.astype(jnp.float32)
        neg_inf = jnp.float32(-jnp.inf)
        row_base = pl.program_id(0) * br

        vfull = (v // 128) * 128
        ya = y[:, :vfull].reshape(br, vfull // 128, 128)
        rem = v - vfull
        if rem:
            yb = jnp.concatenate(
                [y[:, vfull:],
                 jnp.full((br, 128 - rem), neg_inf, jnp.float32)],
                axis=1,
            ).reshape(br, 1, 128)
            ya = jnp.concatenate([ya, yb], axis=1)
        nq = 4 if ya.shape[1] >= 64 else 1
        ntree = -(-ya.shape[1] // (4 * nq)) * (4 * nq)
        npad = ntree - ya.shape[1]
        if npad:
            ypad = jnp.concatenate(
                [ya, jnp.full((br, npad, 128), neg_inf, jnp.float32)],
                axis=1)
        else:
            ypad = ya
        qpad = lax.broadcasted_iota(jnp.int32, (br, ntree, 128), 1)
        w = ntree // nq
        quarters = [
            _per_lane_top4_idx(ypad[:, i * w:(i + 1) * w],
                               qpad[:, i * w:(i + 1) * w])
            for i in range(nq)
        ]
        tops = quarters[0]
        for t in quarters[1:]:
            tops = _merge4(tops, t)
        (m1, q1), (m2, q2), (m3, q3), (m4, q4) = tops
        cand = jnp.concatenate([m1, m2, m3, m4], axis=1)
        qcand = jnp.concatenate([q1, q2, q3, q4], axis=1)
        lane = lax.broadcasted_iota(jnp.int32, (br, 512), 1) & jnp.int32(127)
        col_cand = qcand * jnp.int32(128) + lane
        rmax = jnp.max(m1, axis=1, keepdims=True)

        lo0 = _f32_to_key(jnp.min(m4, axis=1, keepdims=True))
        hi0 = _f32_to_key(rmax)
        t_c = _kth_largest_search(cand, k_self, lo0, hi0)

        one, zero = jnp.float32(1.0), jnp.float32(0.0)
        keep0 = y >= t_c
        cnt_gt = jnp.sum(jnp.where(y > t_c, one, zero), axis=1, keepdims=True)
        cnt_ge = jnp.sum(jnp.where(keep0, one, zero), axis=1, keepdims=True)
        s0 = jnp.where(keep0, jnp.exp(y - rmax), zero)
        denom0 = jnp.sum(s0, axis=1, keepdims=True)
        ccov = jnp.sum(jnp.where(cand >= t_c, one, zero), axis=1,
                       keepdims=True)
        ok = jnp.all((cnt_gt < k_self) & (cnt_ge == ccov))

        def fast():
            probs_ref[...] = s0 * (one / denom0)
            rows = row_base + lax.broadcasted_iota(jnp.int32, (br, 512), 0)
            g = _gumbel_at(rows, col_cand, v)
            z = jnp.where(cand >= t_c, cand + g, neg_inf)
            zmax = jnp.max(z, axis=1, keepdims=True)
            idx_ref[...] = _argmin_idx(z, zmax, col_cand).reshape(br, 1)

        def slow():
            thresh = _kth_largest_search(y, k_self, lo0, hi0)
            keep = y >= thresh
            s = jnp.where(keep, jnp.exp(y - rmax), zero)
            denom = jnp.sum(s, axis=1, keepdims=True)
            probs_ref[...] = s * (one / denom)
            cols = lax.broadcasted_iota(jnp.int32, (br, v), 1)
            rows = row_base + lax.broadcasted_iota(jnp.int32, (br, v), 0)
            g = _gumbel_at(rows, cols, v)
            z = jnp.where(keep, y + g, neg_inf)
            zmax = jnp.max(z, axis=1, keepdims=True)
            idx_ref[...] = _argmin_idx(z, zmax, cols).reshape(br, 1)

        lax.cond(ok, fast, slow)

    return _body


@jax.jit
def kernel(logits, top_k):
    r, v = logits.shape
    br = 8
    tk = jnp.asarray(top_k, jnp.int32).reshape(1)

    grid_spec = pltpu.PrefetchScalarGridSpec(
        num_scalar_prefetch=1,
        grid=(r // br,),
        in_specs=[
            pl.BlockSpec((br, v), lambda i, tk_ref: (i, 0)),
        ],
        out_specs=[
            pl.BlockSpec((br, v), lambda i, tk_ref: (i, 0)),
            pl.BlockSpec((br, 1), lambda i, tk_ref: (i, 0)),
        ],
    )
    probs, idx = pl.pallas_call(
        _make_body(br, v),
        grid_spec=grid_spec,
        out_shape=[
            jax.ShapeDtypeStruct((r, v), jnp.float32),
            jax.ShapeDtypeStruct((r, 1), jnp.int32),
        ],
    )(tk, logits)
    return probs, idx[:, 0]


# speedup vs baseline: 1.9490x; 1.0027x over previous
import functools

import jax
import jax.numpy as jnp
from jax import lax
from jax.experimental import pallas as pl
from jax.experimental.pallas import tpu as pltpu

_TEMPERATURE = 0.8
_KMAX = 50


def _f32_to_key(x):
    i = lax.bitcast_convert_type(x, jnp.int32)
    return i ^ ((i >> 31) & jnp.int32(0x7FFFFFFF))


def _key_to_f32(k):
    i = k ^ ((k >> 31) & jnp.int32(0x7FFFFFFF))
    return lax.bitcast_convert_type(i, jnp.float32)


def _kth_largest_search(data, k_self, lo0, hi0):
    def cond(c):
        lo, hi = c
        return jnp.any(lo < hi)

    def step(c):
        lo, hi = c
        d = lax.bitcast_convert_type(hi - lo, jnp.uint32)
        half = lax.bitcast_convert_type((d + jnp.uint32(1)) >> 1, jnp.int32)
        mid = lo + half
        t = _key_to_f32(mid)
        cnt = jnp.sum(
            jnp.where(data >= t, jnp.float32(1.0), jnp.float32(0.0)),
            axis=1, keepdims=True)
        pred = cnt >= k_self
        return jnp.where(pred, mid, lo), jnp.where(pred, hi, mid - 1)

    lo, _ = lax.while_loop(cond, step, (lo0, hi0))
    return _key_to_f32(lo)


def _sel(c, a, b):
    return jnp.where(c, a, b)


def _per_lane_top4_idx(ypad, qpad):
    n0 = ypad.shape[1]
    h1, h2 = n0 // 2, n0 // 4
    av, bv = ypad[:, :h1], ypad[:, h1:]
    aq, bq = qpad[:, :h1], qpad[:, h1:]
    c = av >= bv
    s1v, s1q = _sel(c, av, bv), _sel(c, aq, bq)
    s2v, s2q = _sel(c, bv, av), _sel(c, bq, aq)
    a1v, a1q = s1v[:, :h2], s1q[:, :h2]
    a2v, a2q = s2v[:, :h2], s2q[:, :h2]
    b1v, b1q = s1v[:, h2:], s1q[:, h2:]
    b2v, b2q = s2v[:, h2:], s2q[:, h2:]
    c1 = a1v >= b1v
    o1v, o1q = _sel(c1, a1v, b1v), _sel(c1, a1q, b1q)
    t1v, t1q = _sel(c1, b1v, a1v), _sel(c1, b1q, a1q)
    c2 = a2v >= b2v
    t2v, t2q = _sel(c2, a2v, b2v), _sel(c2, a2q, b2q)
    o4v, o4q = _sel(c2, b2v, a2v), _sel(c2, b2q, a2q)
    c3 = t1v >= t2v
    o2v, o2q = _sel(c3, t1v, t2v), _sel(c3, t1q, t2q)
    o3v, o3q = _sel(c3, t2v, t1v), _sel(c3, t2q, t1q)
    lists = ((o1v, o1q), (o2v, o2q), (o3v, o3q), (o4v, o4q))
    n = h2
    while n > 1:
        h = n // 2
        a = [(v[:, :h], q[:, :h]) for v, q in lists]
        b = [(v[:, h:2 * h], q[:, h:2 * h]) for v, q in lists]
        carry = None
        if n % 2:
            carry = [(v[:, 2 * h:], q[:, 2 * h:]) for v, q in lists]
        lists = _merge4(a, b)
        if carry is not None:
            lists = tuple(
                (jnp.concatenate([lv, cv], axis=1),
                 jnp.concatenate([lq, cq], axis=1))
                for (lv, lq), (cv, cq) in zip(lists, carry))
            n = h + 1
        else:
            n = h
    return tuple((v[:, 0], q[:, 0]) for v, q in lists)


def _merge4(a, b):
    def hi(x, y):
        c = x[0] >= y[0]
        return _sel(c, x[0], y[0]), _sel(c, x[1], y[1])

    def hilo(x, y):
        c = x[0] >= y[0]
        return ((_sel(c, x[0], y[0]), _sel(c, x[1], y[1])),
                (_sel(c, y[0], x[0]), _sel(c, y[1], x[1])))

    m1 = hi(a[0], b[3])
    m2 = hi(a[1], b[2])
    m3 = hi(a[2], b[1])
    m4 = hi(a[3], b[0])
    p1, p3 = hilo(m1, m3)
    p2, p4 = hilo(m2, m4)
    q1, q2 = hilo(p1, p2)
    q3, q4 = hilo(p3, p4)
    return (q1, q2, q3, q4)


def _gumbel_at(rows, cols, v):
    j = lax.bitcast_convert_type(rows * v + cols, jnp.uint32)
    ks0 = jnp.uint32(0)
    ks1 = jnp.uint32(1)
    ks2 = jnp.uint32(0x1BD11BDA) ^ ks0 ^ ks1
    x0 = jnp.zeros_like(j) + ks0
    x1 = j + ks1

    def rounds(x0, x1, rots):
        for r in rots:
            x0 = x0 + x1
            x1 = (x1 << jnp.uint32(r)) | (x1 >> jnp.uint32(32 - r))
            x1 = x1 ^ x0
        return x0, x1

    r0 = (13, 15, 26, 6)
    r1 = (17, 29, 16, 24)
    x0, x1 = rounds(x0, x1, r0)
    x0, x1 = x0 + ks1, x1 + ks2 + jnp.uint32(1)
    x0, x1 = rounds(x0, x1, r1)
    x0, x1 = x0 + ks2, x1 + ks0 + jnp.uint32(2)
    x0, x1 = rounds(x0, x1, r0)
    x0, x1 = x0 + ks0, x1 + ks1 + jnp.uint32(3)
    x0, x1 = rounds(x0, x1, r1)
    x0, x1 = x0 + ks1, x1 + ks2 + jnp.uint32(4)
    x0, x1 = rounds(x0, x1, r0)
    x0, x1 = x0 + ks2, x1 + ks0 + jnp.uint32(5)
    bits = x0 ^ x1

    fb = (bits >> jnp.uint32(9)) | jnp.uint32(0x3F800000)
    f = lax.bitcast_convert_type(fb, jnp.float32) - jnp.float32(1.0)
    tiny = jnp.float32(jnp.finfo(jnp.float32).tiny)
    u = jnp.maximum(tiny, f * (jnp.float32(1.0) - tiny) + tiny)
    return -jnp.log(-jnp.log(u))


def _argmin_idx(z, zmax, colidx):
    big = jnp.int32(0x7FFFFFFF)
    return jnp.min(jnp.where(z == zmax, colidx, big), axis=1)


def _make_body(br, v):
    def _body(tk_ref, x_ref, probs_ref, idx_ref):
        y = x_ref[...] / jnp.float32(_TEMPERATURE)
        k_sel = jnp.clip(tk_ref[0], 1, _KMAX)
        k_self = k_sel.astype(jnp.float32)
        neg_inf = jnp.float32(-jnp.inf)
        row_base = pl.program_id(0) * br

        vfull = (v // 128) * 128
        ya = y[:, :vfull].reshape(br, vfull // 128, 128)
        rem = v - vfull
        if rem:
            yb = jnp.concatenate(
                [y[:, vfull:],
                 jnp.full((br, 128 - rem), neg_inf, jnp.float32)],
                axis=1,
            ).reshape(br, 1, 128)
            ya = jnp.concatenate([ya, yb], axis=1)
        nq = 4 if ya.shape[1] >= 64 else 1
        ntree = -(-ya.shape[1] // (4 * nq)) * (4 * nq)
        npad = ntree - ya.shape[1]
        if npad:
            ypad = jnp.concatenate(
                [ya, jnp.full((br, npad, 128), neg_inf, jnp.float32)],
                axis=1)
        else:
            ypad = ya
        qpad = lax.broadcasted_iota(jnp.int32, (br, ntree, 128), 1)
        w = ntree // nq
        quarters = [
            _per_lane_top4_idx(ypad[:, i * w:(i + 1) * w],
                               qpad[:, i * w:(i + 1) * w])
            for i in range(nq)
        ]
        tops = quarters[0]
        for t in quarters[1:]:
            tops = _merge4(tops, t)
        (m1, q1), (m2, q2), (m3, q3), (m4, q4) = tops
        cand = jnp.concatenate([m1, m2, m3, m4], axis=1)
        qcand = jnp.concatenate([q1, q2, q3, q4], axis=1)
        lane = lax.broadcasted_iota(jnp.int32, (br, 512), 1) & jnp.int32(127)
        col_cand = qcand * jnp.int32(128) + lane
        rmax = jnp.max(m1, axis=1, keepdims=True)

        lo0 = _f32_to_key(jnp.min(m4, axis=1, keepdims=True))
        hi0 = _f32_to_key(rmax)
        t_c = _kth_largest_search(cand, k_self, lo0, hi0)

        one, zero = jnp.float32(1.0), jnp.float32(0.0)
        keep0 = y >= t_c
        cnt_ge = jnp.sum(jnp.where(keep0, one, zero), axis=1, keepdims=True)
        s0 = jnp.where(keep0, jnp.exp(y - rmax), zero)
        denom0 = jnp.sum(s0, axis=1, keepdims=True)
        ccov = jnp.sum(jnp.where(cand >= t_c, one, zero), axis=1,
                       keepdims=True)
        ok = jnp.all(cnt_ge == ccov)

        def fast():
            probs_ref[...] = s0 * (one / denom0)
            rows = row_base + lax.broadcasted_iota(jnp.int32, (br, 512), 0)
            g = _gumbel_at(rows, col_cand, v)
            z = jnp.where(cand >= t_c, cand + g, neg_inf)
            zmax = jnp.max(z, axis=1, keepdims=True)
            idx_ref[...] = _argmin_idx(z, zmax, col_cand).reshape(br, 1)

        def slow():
            thresh = _kth_largest_search(y, k_self, lo0, hi0)
            keep = y >= thresh
            s = jnp.where(keep, jnp.exp(y - rmax), zero)
            denom = jnp.sum(s, axis=1, keepdims=True)
            probs_ref[...] = s * (one / denom)
            cols = lax.broadcasted_iota(jnp.int32, (br, v), 1)
            rows = row_base + lax.broadcasted_iota(jnp.int32, (br, v), 0)
            g = _gumbel_at(rows, cols, v)
            z = jnp.where(keep, y + g, neg_inf)
            zmax = jnp.max(z, axis=1, keepdims=True)
            idx_ref[...] = _argmin_idx(z, zmax, cols).reshape(br, 1)

        lax.cond(ok, fast, slow)

    return _body


@jax.jit
def kernel(logits, top_k):
    r, v = logits.shape
    br = 8
    tk = jnp.asarray(top_k, jnp.int32).reshape(1)

    grid_spec = pltpu.PrefetchScalarGridSpec(
        num_scalar_prefetch=1,
        grid=(r // br,),
        in_specs=[
            pl.BlockSpec((br, v), lambda i, tk_ref: (i, 0)),
        ],
        out_specs=[
            pl.BlockSpec((br, v), lambda i, tk_ref: (i, 0)),
            pl.BlockSpec((br, 1), lambda i, tk_ref: (i, 0)),
        ],
    )
    probs, idx = pl.pallas_call(
        _make_body(br, v),
        grid_spec=grid_spec,
        out_shape=[
            jax.ShapeDtypeStruct((r, v), jnp.float32),
            jax.ShapeDtypeStruct((r, 1), jnp.int32),
        ],
    )(tk, logits)
    return probs, idx[:, 0]
